# Initial kernel scaffold; baseline (speedup 1.0000x reference)
import jax
import jax.numpy as jnp
from jax.experimental import pallas as pl


def kernel(x, edge_index, edge_weight, W, b):
    raise NotImplementedError("write your pallas kernel here")


# speedup vs baseline: 3.3238x; 3.3238x over previous
import functools

import jax
import jax.numpy as jnp
from jax import lax
from jax.experimental import pallas as pl
from jax.experimental.pallas import tpu as pltpu
from jax.experimental.pallas import tpu_sc as plsc

_NC = 2
_NS = 16
_CHUNK = 128
_LANES = 16


def _sc_aggregate(x, col, row, w, n_nodes):
    n_edges = col.shape[0]
    in_ch = x.shape[1]
    e_tile = n_edges // (_NC * _NS)
    n_chunks = e_tile // _CHUNK
    n_pad = n_nodes + ((-n_nodes) % (_NS * 8))
    rows_per_tile = n_pad // _NS

    mesh = plsc.VectorSubcoreMesh(core_axis_name="c", subcore_axis_name="s")

    @functools.partial(
        pl.kernel,
        mesh=mesh,
        out_type=jax.ShapeDtypeStruct((_NC, n_pad, in_ch), jnp.float32),
        scratch_types=[
            pltpu.VMEM((_CHUNK,), jnp.int32),
            pltpu.VMEM((_CHUNK,), jnp.int32),
            pltpu.VMEM((_CHUNK,), jnp.float32),
            pltpu.VMEM((_CHUNK, in_ch), jnp.float32),
            pltpu.VMEM_SHARED((n_pad, in_ch), jnp.float32),
            pltpu.SemaphoreType.DMA,
        ],
        compiler_params=pltpu.CompilerParams(needs_layout_passes=False),
    )
    def agg_kernel(x_hbm, col_hbm, row_hbm, w_hbm, zero_hbm, out_hbm,
                   col_v, row_v, w_v, rows_v, acc_sh, sem):
        cid = lax.axis_index("c")
        sid = lax.axis_index("s")
        pltpu.sync_copy(zero_hbm,
                        acc_sh.at[pl.ds(sid * rows_per_tile, rows_per_tile)])
        plsc.subcore_barrier()

        tile_base = (cid * _NS + sid) * e_tile

        def chunk_body(c, carry):
            base = tile_base + c * _CHUNK
            pltpu.sync_copy(col_hbm.at[pl.ds(base, _CHUNK)], col_v)
            pltpu.sync_copy(row_hbm.at[pl.ds(base, _CHUNK)], row_v)
            pltpu.sync_copy(w_hbm.at[pl.ds(base, _CHUNK)], w_v)
            pltpu.async_copy(x_hbm.at[col_v], rows_v, sem).wait()

            def scale_body(e, carry2):
                wb = plsc.load_gather(w_v, [jnp.full((_LANES,), e, jnp.int32)])
                for j in range(in_ch // _LANES):
                    sl = rows_v[e, pl.ds(j * _LANES, _LANES)]
                    rows_v[e, pl.ds(j * _LANES, _LANES)] = sl * wb
                return carry2

            lax.fori_loop(0, _CHUNK, scale_body, 0)
            pltpu.sync_copy(rows_v, acc_sh.at[row_v], add=True)
            return carry

        lax.fori_loop(0, n_chunks, chunk_body, 0)
        plsc.subcore_barrier()
        pltpu.sync_copy(acc_sh.at[pl.ds(sid * rows_per_tile, rows_per_tile)],
                        out_hbm.at[cid, pl.ds(sid * rows_per_tile,
                                              rows_per_tile)])

    zero = jnp.zeros((rows_per_tile, in_ch), jnp.float32)
    return agg_kernel(x, col, row, w, zero)


def _tc_linear(parts, W, b, n_nodes):
    in_ch = parts.shape[2]
    out_ch = W.shape[0]
    blk = 1000

    def mm_kernel(p_ref, w_ref, b_ref, o_ref):
        acc = p_ref[0] + p_ref[1]
        o_ref[...] = lax.dot_general(
            acc, w_ref[...], (((1,), (1,)), ((), ())),
            preferred_element_type=jnp.float32) + b_ref[...]

    return pl.pallas_call(
        mm_kernel,
        grid=(n_nodes // blk,),
        in_specs=[
            pl.BlockSpec((2, blk, in_ch), lambda i: (0, i, 0)),
            pl.BlockSpec((out_ch, in_ch), lambda i: (0, 0)),
            pl.BlockSpec((1, out_ch), lambda i: (0, 0)),
        ],
        out_specs=pl.BlockSpec((blk, out_ch), lambda i: (i, 0)),
        out_shape=jax.ShapeDtypeStruct((n_nodes, out_ch), jnp.float32),
    )(parts, W, b.reshape(1, out_ch))


def kernel(x, edge_index, edge_weight, W, b):
    n_nodes = x.shape[0]
    n_edges = edge_weight.shape[0]
    ei = edge_index.astype(jnp.int32)
    epad = (-n_edges) % (_NC * _NS * _CHUNK)
    row = jnp.concatenate([ei[0], jnp.zeros((epad,), jnp.int32)])
    col = jnp.concatenate([ei[1], jnp.zeros((epad,), jnp.int32)])
    w = jnp.concatenate([edge_weight, jnp.zeros((epad,), jnp.float32)])
    parts = _sc_aggregate(x, col, row, w, n_nodes)
    return _tc_linear(parts, W, b, n_nodes)
